# baseline (speedup 1.0000x reference)
import functools

import jax
import jax.numpy as jnp
from jax import lax
from jax.experimental import pallas as pl
from jax.experimental.pallas import tpu as pltpu
from jax.experimental.pallas import tpu_sc as plsc

BATCH = 4096
SEQ = 200
DIM = 32
B = BATCH * SEQ
VOCAB = 1000000

NUM_CORES = 2
NUM_SUBCORES = 16
NW = NUM_CORES * NUM_SUBCORES
B_PER_W = B // NW
CHUNK = 1280
N_CHUNKS = B_PER_W // CHUNK


T_BLK = 4096
T_GRID = (VOCAB + T_BLK - 1) // T_BLK


def _transpose_body(tt_ref, out_ref):
    i = jax.lax.broadcasted_iota(jnp.int32, (DIM, DIM), 0)
    j = jax.lax.broadcasted_iota(jnp.int32, (DIM, DIM), 1)
    eye = (i == j).astype(jnp.float32)
    out_ref[...] = jax.lax.dot_general(
        tt_ref[...], eye, (((0,), (0,)), ((), ())),
        preferred_element_type=jnp.float32,
    )


_transpose_table = pl.pallas_call(
    _transpose_body,
    grid=(T_GRID,),
    in_specs=[pl.BlockSpec((DIM, T_BLK), lambda i: (0, i))],
    out_specs=pl.BlockSpec((T_BLK, DIM), lambda i: (i, 0)),
    out_shape=jax.ShapeDtypeStruct((VOCAB, DIM), jnp.float32),
)


_mesh = plsc.VectorSubcoreMesh(core_axis_name="c", subcore_axis_name="s")


@functools.partial(
    pl.kernel,
    mesh=_mesh,
    out_type=jax.ShapeDtypeStruct((B, DIM), jnp.float32),
    scratch_types=[
        pltpu.VMEM((N_CHUNKS, CHUNK), jnp.int32),
        pltpu.VMEM((CHUNK, DIM), jnp.float32),
        pltpu.VMEM((CHUNK, DIM), jnp.float32),
        pltpu.SemaphoreType.DMA,
        pltpu.SemaphoreType.DMA,
    ],
    compiler_params=pltpu.CompilerParams(use_tc_tiling_on_sc=False),
)
def _gather_kernel(idx_hbm, table_hbm, out_hbm, idx_v, rows0, rows1, sem0, sem1):
    wid = lax.axis_index("s") * NUM_CORES + lax.axis_index("c")
    base = wid * B_PER_W
    pltpu.sync_copy(idx_hbm.at[wid], idx_v)

    rows = (rows0, rows1)
    sems = (sem0, sem1)
    copies = [None, None]
    copies[0] = pltpu.async_copy(table_hbm.at[idx_v.at[0]], rows0, sem0)
    for i in range(N_CHUNKS):
        b = i % 2
        nb = (i + 1) % 2
        if i + 1 < N_CHUNKS:
            copies[nb] = pltpu.async_copy(
                table_hbm.at[idx_v.at[i + 1]], rows[nb], sems[nb]
            )
        copies[b].wait()
        pltpu.sync_copy(rows[b], out_hbm.at[pl.ds(base + i * CHUNK, CHUNK)])


def kernel(x, table):
    table_rm = _transpose_table(table.T)
    idx = x.reshape(NW, N_CHUNKS, CHUNK).astype(jnp.int32)
    out = _gather_kernel(idx, table_rm)
    return out.reshape(BATCH, SEQ, DIM)


# speedup vs baseline: 1.5980x; 1.5980x over previous
import functools

import jax
import jax.numpy as jnp
from jax import lax
from jax.experimental import pallas as pl
from jax.experimental.pallas import tpu as pltpu
from jax.experimental.pallas import tpu_sc as plsc

BATCH = 4096
SEQ = 200
DIM = 32
VOCAB = 1000000
B = BATCH * SEQ

NUM_CORES = 2
NUM_SUBCORES = 16
NW = NUM_CORES * NUM_SUBCORES
B_PER_W = B // NW
CHUNK = 1280
N_CHUNKS = B_PER_W // CHUNK

_mesh = plsc.VectorSubcoreMesh(core_axis_name="c", subcore_axis_name="s")


@functools.partial(
    pl.kernel,
    mesh=_mesh,
    out_type=jax.ShapeDtypeStruct((B, 128), jnp.float32),
    scratch_types=[
        pltpu.VMEM((N_CHUNKS, CHUNK), jnp.int32),
        pltpu.VMEM((CHUNK, DIM), jnp.float32),
        pltpu.VMEM((CHUNK, DIM), jnp.float32),
        pltpu.SemaphoreType.DMA,
        pltpu.SemaphoreType.DMA,
    ],
    compiler_params=pltpu.CompilerParams(use_tc_tiling_on_sc=False),
)
def _gather_kernel(idx_hbm, table_hbm, out_hbm, idx_v, rows0, rows1, sem0, sem1):
    wid = lax.axis_index("s") * NUM_CORES + lax.axis_index("c")
    base = wid * B_PER_W
    pltpu.sync_copy(idx_hbm.at[wid], idx_v)

    rows = (rows0, rows1)
    sems = (sem0, sem1)
    copies = [None, None]
    copies[0] = pltpu.async_copy(table_hbm.at[idx_v.at[0]], rows0, sem0)
    for i in range(N_CHUNKS):
        b = i % 2
        nb = (i + 1) % 2
        if i + 1 < N_CHUNKS:
            copies[nb] = pltpu.async_copy(
                table_hbm.at[idx_v.at[i + 1]], rows[nb], sems[nb]
            )
        copies[b].wait()
        pltpu.sync_copy(
            rows[b],
            out_hbm.at[pl.ds(base + i * CHUNK, CHUNK), pl.ds(0, DIM)],
        )


def kernel(x, table):
    idx = x.reshape(NW, N_CHUNKS, CHUNK).astype(jnp.int32)
    out_pitched = _gather_kernel(idx, table)
    return out_pitched.reshape(BATCH, SEQ, 128)[:, :, :DIM]
